# Initial kernel scaffold; baseline (speedup 1.0000x reference)
import jax
import jax.numpy as jnp
from jax.experimental import pallas as pl


def kernel(sequence, segment_labels, tok_table, seg_table, pe):
    raise NotImplementedError("write your pallas kernel here")


# speedup vs baseline: 1.2399x; 1.2399x over previous
import functools
import math

import jax
import jax.numpy as jnp
from jax import lax
from jax.experimental import pallas as pl
from jax.experimental.pallas import tpu as pltpu
from jax.experimental.pallas import tpu_sc as plsc

B, L, D = 1024, 200, 64
N = B * L
NC, NS, LANES = 2, 16, 16
NW = NC * NS
TPW = N // NW
GS = 128
NG = TPW // GS


def _sc_embed(tok_table, tidx3, aidx3, peseg):
    mesh = plsc.VectorSubcoreMesh(core_axis_name="c", subcore_axis_name="s")

    @functools.partial(
        pl.kernel,
        mesh=mesh,
        compiler_params=pltpu.CompilerParams(use_tc_tiling_on_sc=False),
        out_type=jax.ShapeDtypeStruct((N, D), jnp.float32),
        scratch_types=[
            pltpu.VMEM((NG, GS), jnp.int32),
            pltpu.VMEM((NG, GS), jnp.int32),
            pltpu.VMEM((GS, D), jnp.float32),
            pltpu.VMEM((GS, D), jnp.float32),
            pltpu.SemaphoreType.DMA,
            pltpu.SemaphoreType.DMA,
        ],
    )
    def k(tok_hbm, tidx_hbm, aidx_hbm, peseg_hbm, out_hbm,
          tidx_v, aidx_v, tok_v, add_v, sem_t, sem_a):
        wid = lax.axis_index("s") * NC + lax.axis_index("c")
        pltpu.sync_copy(tidx_hbm.at[wid], tidx_v)
        pltpu.sync_copy(aidx_hbm.at[wid], aidx_v)

        def group(g, carry):
            cp_t = pltpu.async_copy(tok_hbm.at[tidx_v.at[g]], tok_v, sem_t)
            cp_a = pltpu.async_copy(peseg_hbm.at[aidx_v.at[g]], add_v, sem_a)
            cp_t.wait()
            cp_a.wait()

            def row(r, c2):
                for c in range(D // LANES):
                    sl = pl.ds(c * LANES, LANES)
                    tok_v[r, sl] = tok_v[r, sl] + add_v[r, sl]
                return c2

            lax.fori_loop(0, GS, row, 0)
            pltpu.sync_copy(tok_v, out_hbm.at[pl.ds(wid * TPW + g * GS, GS)])
            return carry

        lax.fori_loop(0, NG, group, 0)

    return k(tok_table, tidx3, aidx3, peseg)


def kernel(sequence, segment_labels, tok_table, seg_table, pe):
    tidx3 = sequence.astype(jnp.int32).reshape(NW, NG, GS)
    l_pos = jnp.arange(L, dtype=jnp.int32)
    aidx3 = (segment_labels.astype(jnp.int32) * L + l_pos[None, :]).reshape(NW, NG, GS)
    peseg = (seg_table[:, None, :] + pe[0, :L, :][None, :, :]).reshape(3 * L, D)
    out = _sc_embed(tok_table, tidx3, aidx3, peseg)
    return out.reshape(B, L, D)
